# baseline (speedup 1.0000x reference)
import functools

import jax
import jax.numpy as jnp
from jax import lax
from jax.experimental import pallas as pl
from jax.experimental.pallas import tpu as pltpu
from jax.experimental.pallas import tpu_sc as plsc

N_U = 50000
N_M = 10000
E = 500000
L = 100000
D = 128
H = 128

NC = 2
NS = 16
NW = NC * NS

EPW = 16384
EPAD = EPW * NW
KU = 1024
NBU = EPW // KU

NUP = 50048
NMP = 10112

KD = 256
DBLK = 13
LPW = KD * DBLK
LPAD = LPW * NW

_MESH = plsc.VectorSubcoreMesh(core_axis_name="c", subcore_axis_name="s")


def _zero_acc_slice(zbuf, acc, start, znum, kmax):
    off = 0
    while off < znum:
        n = min(kmax, znum - off)
        pltpu.sync_copy(zbuf.at[pl.ds(0, n)], acc.at[pl.ds(start + off, n)])
        off += n


@functools.cache
def _make_agg(n_src: int, n_dst_p: int, w: int, nch: int):
    Z = n_dst_p // NS

    def body(tables, gidx, sidx, zo, out, gv, sv, rows, zbuf, acc, sem):
        c = lax.axis_index("c")
        s = lax.axis_index("s")
        pltpu.sync_copy(gidx.at[c, s], gv)
        pltpu.sync_copy(sidx.at[c, s], sv)
        pltpu.sync_copy(zo.at[0], zbuf)
        for cc in range(nch + 1):
            _zero_acc_slice(zbuf, acc, s * Z, Z, KU)
            plsc.subcore_barrier()
            if cc < nch:
                def blk(b, _):
                    pltpu.async_copy(
                        tables.at[cc].at[gv.at[pl.ds(b * KU, KU)]], rows,
                        sem).wait()
                    pltpu.sync_copy(rows, acc.at[sv.at[pl.ds(b * KU, KU)]],
                                    add=True)
                    return 0

                lax.fori_loop(0, NBU, blk, 0)
            else:
                pltpu.sync_copy(zo.at[1], rows)

                def blkc(b, _):
                    pltpu.sync_copy(rows, acc.at[sv.at[pl.ds(b * KU, KU)]],
                                    add=True)
                    return 0

                lax.fori_loop(0, NBU, blkc, 0)
            plsc.subcore_barrier()
            pltpu.sync_copy(acc.at[pl.ds(s * Z, Z)],
                            out.at[c, cc, pl.ds(s * Z, Z), :])
            plsc.subcore_barrier()

    return pl.kernel(
        body,
        out_type=jax.ShapeDtypeStruct((NC, nch + 1, n_dst_p, w), jnp.float32),
        mesh=_MESH,
        scratch_types=[
            pltpu.VMEM((EPW,), jnp.int32),
            pltpu.VMEM((EPW,), jnp.int32),
            pltpu.VMEM((KU, w), jnp.float32),
            pltpu.VMEM((KU, w), jnp.float32),
            pltpu.VMEM_SHARED((n_dst_p, w), jnp.float32),
            pltpu.SemaphoreType.DMA,
        ],
        compiler_params=pltpu.CompilerParams(use_tc_tiling_on_sc=False),
    )


@functools.cache
def _make_label_gather():
    def body(zu, zm, iu_h, im_h, ou, om, iu, im, rows, sem):
        c = lax.axis_index("c")
        s = lax.axis_index("s")
        wid = c * NS + s
        pltpu.sync_copy(iu_h.at[c, s], iu)
        pltpu.sync_copy(im_h.at[c, s], im)
        base = wid * LPW

        def blk(b, _):
            pltpu.async_copy(zu.at[iu.at[pl.ds(b * KD, KD)]], rows,
                             sem).wait()
            pltpu.sync_copy(rows, ou.at[pl.ds(base + b * KD, KD), :])
            pltpu.async_copy(zm.at[im.at[pl.ds(b * KD, KD)]], rows,
                             sem).wait()
            pltpu.sync_copy(rows, om.at[pl.ds(base + b * KD, KD), :])
            return 0

        lax.fori_loop(0, DBLK, blk, 0)

    return pl.kernel(
        body,
        out_type=[
            jax.ShapeDtypeStruct((LPAD, 128), jnp.float32),
            jax.ShapeDtypeStruct((LPAD, 128), jnp.float32),
        ],
        mesh=_MESH,
        scratch_types=[
            pltpu.VMEM((LPW,), jnp.int32),
            pltpu.VMEM((LPW,), jnp.int32),
            pltpu.VMEM((KD, 128), jnp.float32),
            pltpu.SemaphoreType.DMA,
        ],
    )


_RB = 512


def _make_combine1_body(w, nch):
    def body(pref, xref, wl, wr, bl, href, h0, h1, h2, h3):
        p = pref[...]
        agg = jnp.concatenate([p[0, cc] + p[1, cc] for cc in range(nch)],
                              axis=1)
        cnt = p[0, nch, :, 0:1] + p[1, nch, :, 0:1]
        inv = 1.0 / jnp.maximum(cnt, 1.0)
        h = (jnp.dot(agg * inv, wl[...], preferred_element_type=jnp.float32)
             + jnp.dot(xref[...], wr[...], preferred_element_type=jnp.float32)
             + bl[...])
        h = jnp.maximum(h, 0.0)
        href[...] = h
        for cc, hc in enumerate((h0, h1, h2, h3)):
            hc[...] = h[:, 32 * cc:32 * (cc + 1)]
    return body


def _make_combine2_body(w, nch):
    def body(pref, xref, wl, wr, bl, zref):
        p = pref[...]
        agg = jnp.concatenate([p[0, cc] + p[1, cc] for cc in range(nch)],
                              axis=1)
        cnt = p[0, nch, :, 0:1] + p[1, nch, :, 0:1]
        inv = 1.0 / jnp.maximum(cnt, 1.0)
        zref[...] = (jnp.dot(agg * inv, wl[...],
                             preferred_element_type=jnp.float32)
                     + jnp.dot(xref[...], wr[...],
                               preferred_element_type=jnp.float32)
                     + bl[...])
    return body


def _wspec():
    return pl.BlockSpec((128, 128), lambda r: (0, 0))


def _bspec():
    return pl.BlockSpec((1, 128), lambda r: (0, 0))


def _combine1(P, x, Wl, Wr, bl, *, n, npad, w, nch):
    grid = (npad + _RB - 1) // _RB
    return pl.pallas_call(
        _make_combine1_body(w, nch),
        grid=(grid,),
        in_specs=[
            pl.BlockSpec((NC, nch + 1, _RB, w), lambda r: (0, 0, r, 0)),
            pl.BlockSpec((_RB, 128), lambda r: (r, 0)),
            _wspec(), _wspec(), _bspec(),
        ],
        out_specs=[pl.BlockSpec((_RB, 128), lambda r: (r, 0))]
        + [pl.BlockSpec((_RB, 32), lambda r: (r, 0))] * 4,
        out_shape=[jax.ShapeDtypeStruct((n, 128), jnp.float32)]
        + [jax.ShapeDtypeStruct((n, 32), jnp.float32)] * 4,
    )(P, x, Wl, Wr, bl)


def _combine2(P, x, Wl, Wr, bl, *, n, npad, w, nch):
    grid = (npad + _RB - 1) // _RB
    return pl.pallas_call(
        _make_combine2_body(w, nch),
        grid=(grid,),
        in_specs=[
            pl.BlockSpec((NC, nch + 1, _RB, w), lambda r: (0, 0, r, 0)),
            pl.BlockSpec((_RB, 128), lambda r: (r, 0)),
            _wspec(), _wspec(), _bspec(),
        ],
        out_specs=pl.BlockSpec((_RB, 128), lambda r: (r, 0)),
        out_shape=jax.ShapeDtypeStruct((n, 128), jnp.float32),
    )(P, x, Wl, Wr, bl)


def _decoder_body(zuref, zmref, w1a, w1b, b1, w2, b2, oref):
    h = (jnp.dot(zuref[...], w1a[...], preferred_element_type=jnp.float32)
         + jnp.dot(zmref[...], w1b[...], preferred_element_type=jnp.float32)
         + b1[...])
    h = jnp.maximum(h, 0.0)
    oref[...] = jnp.sum(h * w2[...], axis=1, keepdims=True) + b2[...]


def _decoder(zug, zmg, Wd1, bd1, Wd2, bd2):
    grid = LPAD // _RB
    return pl.pallas_call(
        _decoder_body,
        grid=(grid,),
        in_specs=[
            pl.BlockSpec((_RB, 128), lambda r: (r, 0)),
            pl.BlockSpec((_RB, 128), lambda r: (r, 0)),
            _wspec(), _wspec(), _bspec(), _bspec(),
            pl.BlockSpec((1, 1), lambda r: (0, 0)),
        ],
        out_specs=pl.BlockSpec((_RB, 1), lambda r: (r, 0)),
        out_shape=jax.ShapeDtypeStruct((LPAD, 1), jnp.float32),
    )(zug, zmg, Wd1[:128], Wd1[128:], bd1.reshape(1, 128),
      Wd2.reshape(1, 128), bd2.reshape(1, 1))


def _prep_edges(ix, padval):
    pad = jnp.full((EPAD - E,), padval, jnp.int32)
    return jnp.concatenate([ix, pad]).reshape(NC, NS, EPW)


def _prep_labels(ix):
    pad = jnp.zeros((LPAD - L,), jnp.int32)
    return jnp.concatenate([ix, pad]).reshape(NC, NS, LPW)


def _chunks(x, w):
    nch = 128 // w
    return jnp.stack([x[:, w * cc:w * (cc + 1)] for cc in range(nch)])


def kernel(x_user, x_movie, edge_src, edge_dst, label_src, label_dst,
           Wl1u, bl1u, Wr1u, Wl1m, bl1m, Wr1m,
           Wl2u, bl2u, Wr2u, Wl2m, bl2m, Wr2m,
           Wd1, bd1, Wd2, bd2):
    eg_d_u = _prep_edges(edge_dst, 0)
    es_u = _prep_edges(edge_src, N_U)
    eg_s_m = _prep_edges(edge_src, 0)
    es_m = _prep_edges(edge_dst, N_M)

    xm_c = _chunks(x_movie, 8)
    xu_c = _chunks(x_user, 32)

    agg_u = _make_agg(N_M, NUP, 8, 16)
    agg_m = _make_agg(N_U, NMP, 32, 4)

    zo8 = jnp.stack([jnp.zeros((KU, 8), jnp.float32),
                     jnp.ones((KU, 8), jnp.float32)])
    zo32 = jnp.stack([jnp.zeros((KU, 32), jnp.float32),
                      jnp.ones((KU, 32), jnp.float32)])
    Pu1 = agg_u(xm_c, eg_d_u, es_u, zo8)
    tok = (Pu1[0, 0, 0, 0] * 0.0).astype(jnp.int32)
    Pm1 = agg_m(xu_c, eg_s_m, es_m + tok, zo32)

    h_u, hu0, hu1, hu2, hu3 = _combine1(
        Pu1, x_user, Wl1u, Wr1u, bl1u.reshape(1, 128), n=N_U, npad=NUP,
        w=8, nch=16)
    h_m, hm0, hm1, hm2, hm3 = _combine1(
        Pm1, x_movie, Wl1m, Wr1m, bl1m.reshape(1, 128), n=N_M, npad=NMP,
        w=32, nch=4)

    hm_c = _chunks(h_m, 8)
    tok1 = (Pm1[0, 0, 0, 0] * 0.0).astype(jnp.int32)
    Pu2 = agg_u(hm_c, eg_d_u, es_u + tok1, zo8)
    tok2 = (Pu2[0, 0, 0, 0] * 0.0).astype(jnp.int32)
    Pm2 = agg_m(jnp.stack([hu0, hu1, hu2, hu3]), eg_s_m,
                es_m + tok2, zo32)

    z_u = _combine2(Pu2, h_u, Wl2u, Wr2u, bl2u.reshape(1, 128),
                    n=N_U, npad=NUP, w=8, nch=16)
    z_m = _combine2(Pm2, h_m, Wl2m, Wr2m, bl2m.reshape(1, 128),
                    n=N_M, npad=NMP, w=32, nch=4)

    zug, zmg = _make_label_gather()(z_u, z_m, _prep_labels(label_src),
                                    _prep_labels(label_dst))
    o = _decoder(zug, zmg, Wd1, bd1, Wd2, bd2)
    return o[:L, 0]
